# Initial kernel scaffold; baseline (speedup 1.0000x reference)
import jax
import jax.numpy as jnp
from jax.experimental import pallas as pl


def kernel(x, edge_index, t, emb_pcs, W_enc, b_enc, W_f, Wt, bt, m1W1, m1b1, m1W2, m1b2, m2W1, m2b1, m2W2, m2b2, m3W1, m3b1, m3W2, m3b2):
    raise NotImplementedError("write your pallas kernel here")


# speedup vs baseline: 1.1537x; 1.1537x over previous
import functools
import math

import jax
import jax.numpy as jnp
from jax import lax
from jax.experimental import pallas as pl
from jax.experimental.pallas import tpu as pltpu
from jax.experimental.pallas import tpu_sc as plsc

N = 10000
E = 320000
F = 128
SIGMA = 25.0

NC = 2
NS = 16
L = 16

HN = 5000
HNP = 5120
NPAD = 2 * HNP
EH = E // 2
CE = 640
NCHUNK = EH // CE
EPW = E // (NC * NS)
CG = 80
NGCHUNK = EPW // CG

_mesh = plsc.VectorSubcoreMesh(
    core_axis_name="c", subcore_axis_name="s", num_cores=NC, num_subcores=NS)

NEG_INF = float("-inf")


def _pre_body(x_ref, t_ref, wf_ref, wt_ref, bt_ref, wenc_ref, benc_ref,
              temb_ref, h0_ref):
    t = t_ref[...]
    wf = wf_ref[...]
    proj = t * wf * jnp.float32(2.0 * math.pi)
    temb0 = jnp.concatenate([jnp.sin(proj), jnp.cos(proj)], axis=-1)
    z = jnp.dot(temb0, wt_ref[...], preferred_element_type=jnp.float32)
    z = z + bt_ref[...]
    temb_ref[...] = z * jax.nn.sigmoid(z)
    h0 = jnp.dot(x_ref[...], wenc_ref[...], preferred_element_type=jnp.float32)
    h0_ref[...] = h0 + benc_ref[...]


def _node_body(prep, ga_ref, gb_ref, temb_ref, emb_ref,
               cg_ref, ct_ref, ce_ref, bg_ref, bt_ref, be_ref, b1_ref,
               p_ref, q_ref):
    if prep:
        m = jnp.maximum(ga_ref[...], gb_ref[...])
        m = jnp.where(m > NEG_INF, m, 0.0)
        g = jnp.maximum(m, 0.0)
    else:
        g = ga_ref[...]
    temb = temb_ref[...]
    emb = emb_ref[...]
    dot = lambda a, w: jnp.dot(a, w[...], preferred_element_type=jnp.float32)
    p_ref[...] = (dot(g, cg_ref) + dot(temb, ct_ref) + dot(emb, ce_ref)
                  + b1_ref[...])
    q_ref[...] = dot(g, bg_ref) + dot(temb, bt_ref) + dot(emb, be_ref)


def _msg_body(pre_ref, w2_ref, b2_ref, m_ref):
    a = jnp.maximum(pre_ref[...], 0.0)
    m_ref[...] = (jnp.dot(a, w2_ref[...], preferred_element_type=jnp.float32)
                  + b2_ref[...])


def _final_body(pa_ref, pb_ref, t_ref, out_ref):
    m = jnp.maximum(pa_ref[...], pb_ref[...])
    m = jnp.where(m > NEG_INF, m, 0.0)
    log_sigma = jnp.float32(math.log(SIGMA))
    t = t_ref[...]
    var = (jnp.exp(2.0 * t * log_sigma) - 1.0) / (2.0 * log_sigma)
    std = jnp.sqrt(var)
    out_ref[...] = m / (std + 1e-7)


_NB = 1000
_EB = 1000


def _full(shape):
    nd = len(shape)
    return pl.BlockSpec(shape, lambda i: (0,) * nd)


def _rows(block_shape):
    return pl.BlockSpec(block_shape, lambda i: (i,) + (0,) * (len(block_shape) - 1))


def _tc_pre(x, t, wf, wt, bt, wenc, benc):
    return pl.pallas_call(
        _pre_body,
        grid=(N // _NB,),
        in_specs=[_rows((_NB, F)), _rows((_NB, 1)), _full((1, F // 2)),
                  _full((F, F)), _full((1, F)), _full((F, F)), _full((1, F))],
        out_specs=[_rows((_NB, F)), _rows((_NB, F))],
        out_shape=[jax.ShapeDtypeStruct((N, F), jnp.float32),
                   jax.ShapeDtypeStruct((N, F), jnp.float32)],
    )(x, t, wf, wt, bt, wenc, benc)


def _tc_node(prep, ga, gb, temb, emb, cg, ct, ce, bg, bt, be, b1):
    return pl.pallas_call(
        functools.partial(_node_body, prep),
        grid=(N // _NB,),
        in_specs=[_rows((_NB, F))] * 4 + [_full((F, F))] * 6 + [_full((1, F))],
        out_specs=[_rows((_NB, F)), _rows((_NB, F))],
        out_shape=[jax.ShapeDtypeStruct((N, F), jnp.float32),
                   jax.ShapeDtypeStruct((N, F), jnp.float32)],
    )(ga, gb, temb, emb, cg, ct, ce, bg, bt, be, b1)


def _tc_msg(pre, w2, b2):
    return pl.pallas_call(
        _msg_body,
        grid=(E // _EB,),
        in_specs=[_rows((_EB, F)), _full((F, F)), _full((1, F))],
        out_specs=_rows((_EB, F)),
        out_shape=jax.ShapeDtypeStruct((E, F), jnp.float32),
    )(pre, w2, b2)


def _tc_final(pa, pb, t):
    return pl.pallas_call(
        _final_body,
        grid=(N // _NB,),
        in_specs=[_rows((_NB, F)), _rows((_NB, F)), _rows((_NB, 1))],
        out_specs=_rows((_NB, F)),
        out_shape=jax.ShapeDtypeStruct((N, F), jnp.float32),
    )(pa, pb, t)


def _gather_body(p_hbm, q_hbm, dst_hbm, src_hbm, pre_hbm,
                 dstb, srcb, dbuf, sbuf):
    wid = lax.axis_index("s") * NC + lax.axis_index("c")
    base = wid * EPW
    pltpu.sync_copy(dst_hbm.at[pl.ds(base, EPW)], dstb)
    pltpu.sync_copy(src_hbm.at[pl.ds(base, EPW)], srcb)

    def chunk(g, carry):
        off = g * CG
        pltpu.sync_copy(p_hbm.at[dstb.at[pl.ds(off, CG)]], dbuf)
        pltpu.sync_copy(q_hbm.at[srcb.at[pl.ds(off, CG)]], sbuf)

        def row(r, c2):
            for j in range(F // L):
                sl = pl.ds(j * L, L)
                dbuf[r, sl] = dbuf[r, sl] + sbuf[r, sl]
            return c2

        lax.fori_loop(0, CG, row, 0)
        pltpu.sync_copy(dbuf, pre_hbm.at[pl.ds(base + off, CG), :])
        return carry

    lax.fori_loop(0, NGCHUNK, chunk, 0)


def _sc_gather(p, q, dst, src):
    kern = pl.kernel(
        _gather_body,
        out_type=jax.ShapeDtypeStruct((E, F), jnp.float32),
        mesh=_mesh,
        scratch_types=[
            pltpu.VMEM((EPW,), jnp.int32),
            pltpu.VMEM((EPW,), jnp.int32),
            pltpu.VMEM((CG, F), jnp.float32),
            pltpu.VMEM((CG, F), jnp.float32),
        ],
    )
    return kern(p, q, dst, src)


def _scatter_body(dst_hbm, mv_hbm, part_hbm, acc, dstb, mbuf, gidx, oidx):
    s = lax.axis_index("c")
    t = lax.axis_index("s")
    nh = t // 8
    fb = t % 8
    lanes = lax.iota(jnp.int32, L)

    neg = jnp.full((L,), NEG_INF, jnp.float32)

    def initrow(r, c):
        acc[r, :] = neg
        return c

    lax.fori_loop(0, HNP, initrow, 0)

    ebase = s * EH

    def chunk(g, carry):
        e0 = ebase + g * CE
        pltpu.sync_copy(dst_hbm.at[pl.ds(e0, CE)], dstb)

        def fill(j, c2):
            gidx[pl.ds(j * L, L)] = (e0 + j * L + lanes) * 8 + fb
            return c2

        lax.fori_loop(0, CE // L, fill, 0)
        for k in range(CE // 128):
            pltpu.sync_copy(mv_hbm.at[gidx.at[pl.ds(k * 128, 128)]],
                            mbuf.at[pl.ds(k * 128, 128)])

        def edge16(j, c2):
            dvec = dstb[pl.ds(j * L, L)]
            base16 = j * L
            for i in range(L):
                d = dvec[i]
                loc = d - nh * HN
                inr = jnp.logical_and(loc >= 0, loc < HN)
                lc = jnp.where(inr, loc, 0)
                pen = jnp.where(inr, jnp.float32(0.0), jnp.float32(NEG_INF))
                row = acc[lc]
                m = mbuf[base16 + i]
                acc[lc] = jnp.maximum(row, m + pen)
            return c2

        lax.fori_loop(0, CE // L, edge16, 0)
        return carry

    lax.fori_loop(0, NCHUNK, chunk, 0)

    obase = s * (NPAD * 8) + (nh * HNP) * 8 + fb

    def wb(g, carry):
        def ofill(j, c2):
            oidx[g, pl.ds(j * L, L)] = obase + (g * 128 + j * L + lanes) * 8
            return c2

        lax.fori_loop(0, 128 // L, ofill, 0)
        pltpu.sync_copy(acc.at[pl.ds(g * 128, 128)], part_hbm.at[oidx.at[g]])
        return carry

    lax.fori_loop(0, HNP // 128, wb, 0)


def _sc_scatter(dst, mv):
    kern = pl.kernel(
        _scatter_body,
        out_type=jax.ShapeDtypeStruct((2 * NPAD * 8, L), jnp.float32),
        mesh=_mesh,
        scratch_types=[
            pltpu.VMEM((HNP, L), jnp.float32),
            pltpu.VMEM((CE,), jnp.int32),
            pltpu.VMEM((CE, L), jnp.float32),
            pltpu.VMEM((CE,), jnp.int32),
            pltpu.VMEM((HNP // 128, 128), jnp.int32),
        ],
        compiler_params=pltpu.CompilerParams(use_tc_tiling_on_sc=False),
    )
    return kern(dst, mv)


def _segment_max_parts(dst, m):
    mv = m.reshape(E * 8, L)
    partv = _sc_scatter(dst, mv)
    part = partv.reshape(2, NPAD, F)
    pa = jnp.concatenate([part[0, :HN], part[0, HNP:HNP + HN]], axis=0)
    pb = jnp.concatenate([part[1, :HN], part[1, HNP:HNP + HN]], axis=0)
    return pa, pb


def kernel(x, edge_index, t, emb_pcs, W_enc, b_enc, W_f, Wt, bt,
           m1W1, m1b1, m1W2, m1b2, m2W1, m2b1, m2W2, m2b2,
           m3W1, m3b1, m3W2, m3b2):
    src = edge_index[0].astype(jnp.int32)
    dst = edge_index[1].astype(jnp.int32)

    def split(w1):
        w1a, w1b = w1[:3 * F], w1[3 * F:]
        c = w1a - w1b
        return ((c[:F], c[F:2 * F], c[2 * F:]),
                (w1b[:F], w1b[F:2 * F], w1b[2 * F:]))

    temb, h0 = _tc_pre(x, t, W_f.reshape(1, F // 2), Wt, bt.reshape(1, F),
                       W_enc, b_enc.reshape(1, F))

    ga, gb = h0, h0
    prep = False
    for (w1, b1, w2, b2) in ((m1W1, m1b1, m1W2, m1b2),
                             (m2W1, m2b1, m2W2, m2b2),
                             (m3W1, m3b1, m3W2, m3b2)):
        (cg, ct, ce), (bg, bt_, be) = split(w1)
        p, q = _tc_node(prep, ga, gb, temb, emb_pcs,
                        cg, ct, ce, bg, bt_, be, b1.reshape(1, F))
        pre = _sc_gather(p, q, dst, src)
        m = _tc_msg(pre, w2, b2.reshape(1, F))
        ga, gb = _segment_max_parts(dst, m)
        prep = True

    return _tc_final(ga, gb, t)


# speedup vs baseline: 2.1214x; 1.8387x over previous
import functools
import math

import jax
import jax.numpy as jnp
from jax import lax
from jax.experimental import pallas as pl
from jax.experimental.pallas import tpu as pltpu
from jax.experimental.pallas import tpu_sc as plsc

N = 10000
E = 320000
F = 128
SIGMA = 25.0

NC = 2
NS = 16
L = 16

HN = 5000
HNP = 5120
NPAD = 2 * HNP
EH = E // 2
CE = 640
NCHUNK = EH // CE
EPW = E // (NC * NS)
CG = 80
NGCHUNK = EPW // CG

_mesh = plsc.VectorSubcoreMesh(
    core_axis_name="c", subcore_axis_name="s", num_cores=NC, num_subcores=NS)

NEG_INF = float("-inf")


def _pre_body(x_ref, t_ref, wf_ref, wt_ref, bt_ref, wenc_ref, benc_ref,
              temb_ref, h0_ref):
    t = t_ref[...]
    wf = wf_ref[...]
    proj = t * wf * jnp.float32(2.0 * math.pi)
    temb0 = jnp.concatenate([jnp.sin(proj), jnp.cos(proj)], axis=-1)
    z = jnp.dot(temb0, wt_ref[...], preferred_element_type=jnp.float32)
    z = z + bt_ref[...]
    temb_ref[...] = z * jax.nn.sigmoid(z)
    h0 = jnp.dot(x_ref[...], wenc_ref[...], preferred_element_type=jnp.float32)
    h0_ref[...] = h0 + benc_ref[...]


def _node_body(prep, ga_ref, gb_ref, temb_ref, emb_ref,
               cg_ref, ct_ref, ce_ref, bg_ref, bt_ref, be_ref, b1_ref,
               p_ref, q_ref):
    if prep:
        m = jnp.maximum(ga_ref[...], gb_ref[...])
        m = jnp.where(m > NEG_INF, m, 0.0)
        g = jnp.maximum(m, 0.0)
    else:
        g = ga_ref[...]
    temb = temb_ref[...]
    emb = emb_ref[...]
    dot = lambda a, w: jnp.dot(a, w[...], preferred_element_type=jnp.float32)
    p_ref[...] = (dot(g, cg_ref) + dot(temb, ct_ref) + dot(emb, ce_ref)
                  + b1_ref[...])
    q_ref[...] = dot(g, bg_ref) + dot(temb, bt_ref) + dot(emb, be_ref)


def _msg_body(pre_ref, w2_ref, b2_ref, m_ref):
    a = jnp.maximum(pre_ref[...], 0.0)
    m_ref[...] = (jnp.dot(a, w2_ref[...], preferred_element_type=jnp.float32)
                  + b2_ref[...])


def _final_body(pa_ref, pb_ref, t_ref, out_ref):
    m = jnp.maximum(pa_ref[...], pb_ref[...])
    m = jnp.where(m > NEG_INF, m, 0.0)
    log_sigma = jnp.float32(math.log(SIGMA))
    t = t_ref[...]
    var = (jnp.exp(2.0 * t * log_sigma) - 1.0) / (2.0 * log_sigma)
    std = jnp.sqrt(var)
    out_ref[...] = m / (std + 1e-7)


_NB = 1000
_EB = 1000


def _full(shape):
    nd = len(shape)
    return pl.BlockSpec(shape, lambda i: (0,) * nd)


def _rows(block_shape):
    return pl.BlockSpec(block_shape, lambda i: (i,) + (0,) * (len(block_shape) - 1))


def _tc_pre(x, t, wf, wt, bt, wenc, benc):
    return pl.pallas_call(
        _pre_body,
        grid=(N // _NB,),
        in_specs=[_rows((_NB, F)), _rows((_NB, 1)), _full((1, F // 2)),
                  _full((F, F)), _full((1, F)), _full((F, F)), _full((1, F))],
        out_specs=[_rows((_NB, F)), _rows((_NB, F))],
        out_shape=[jax.ShapeDtypeStruct((N, F), jnp.float32),
                   jax.ShapeDtypeStruct((N, F), jnp.float32)],
    )(x, t, wf, wt, bt, wenc, benc)


def _tc_node(prep, ga, gb, temb, emb, cg, ct, ce, bg, bt, be, b1):
    return pl.pallas_call(
        functools.partial(_node_body, prep),
        grid=(N // _NB,),
        in_specs=[_rows((_NB, F))] * 4 + [_full((F, F))] * 6 + [_full((1, F))],
        out_specs=[_rows((_NB, F)), _rows((_NB, F))],
        out_shape=[jax.ShapeDtypeStruct((N, F), jnp.float32),
                   jax.ShapeDtypeStruct((N, F), jnp.float32)],
    )(ga, gb, temb, emb, cg, ct, ce, bg, bt, be, b1)


def _tc_msg(pre, w2, b2):
    return pl.pallas_call(
        _msg_body,
        grid=(E // _EB,),
        in_specs=[_rows((_EB, F)), _full((F, F)), _full((1, F))],
        out_specs=_rows((_EB, F)),
        out_shape=jax.ShapeDtypeStruct((E, F), jnp.float32),
    )(pre, w2, b2)


def _tc_final(pa, pb, t):
    return pl.pallas_call(
        _final_body,
        grid=(N // _NB,),
        in_specs=[_rows((_NB, F)), _rows((_NB, F)), _rows((_NB, 1))],
        out_specs=_rows((_NB, F)),
        out_shape=jax.ShapeDtypeStruct((N, F), jnp.float32),
    )(pa, pb, t)


def _gather_body(p_hbm, q_hbm, dst_hbm, src_hbm, pre_hbm,
                 dstb, srcb, dbuf, sbuf):
    wid = lax.axis_index("s") * NC + lax.axis_index("c")
    base = wid * EPW
    pltpu.sync_copy(dst_hbm.at[pl.ds(base, EPW)], dstb)
    pltpu.sync_copy(src_hbm.at[pl.ds(base, EPW)], srcb)

    def chunk(g, carry):
        off = g * CG
        pltpu.sync_copy(p_hbm.at[dstb.at[pl.ds(off, CG)]], dbuf)
        pltpu.sync_copy(q_hbm.at[srcb.at[pl.ds(off, CG)]], sbuf)

        def row(r, c2):
            for j in range(F // L):
                sl = pl.ds(j * L, L)
                dbuf[r, sl] = dbuf[r, sl] + sbuf[r, sl]
            return c2

        lax.fori_loop(0, CG, row, 0)
        pltpu.sync_copy(dbuf, pre_hbm.at[pl.ds(base + off, CG), :])
        return carry

    lax.fori_loop(0, NGCHUNK, chunk, 0)


def _sc_gather(p, q, dst, src):
    kern = pl.kernel(
        _gather_body,
        out_type=jax.ShapeDtypeStruct((E, F), jnp.float32),
        mesh=_mesh,
        scratch_types=[
            pltpu.VMEM((EPW,), jnp.int32),
            pltpu.VMEM((EPW,), jnp.int32),
            pltpu.VMEM((CG, F), jnp.float32),
            pltpu.VMEM((CG, F), jnp.float32),
        ],
    )
    return kern(p, q, dst, src)


def _scatter_body(dst_hbm, mv_hbm, part_hbm, acc,
                  dstb0, dstb1, mbuf0, mbuf1, gidx, oidx, sem0, sem1):
    s = lax.axis_index("c")
    fb = lax.axis_index("s")
    lanes = lax.iota(jnp.int32, L)
    fsel = jnp.bitwise_and(lanes, 7)
    lanehigh = jnp.right_shift(lanes, 3)
    swapidx = jnp.bitwise_xor(lanes, 8)

    neg = jnp.full((L,), NEG_INF, jnp.float32)

    def initrow(r, c):
        plsc.store_scatter(acc, (2 * r + lanehigh, fsel), neg)
        return c

    lax.fori_loop(0, NPAD // 2, initrow, 0)

    ebase = s * EH
    dstbs = (dstb0, dstb1)
    mbufs = (mbuf0, mbuf1)
    sems = (sem0, sem1)

    fbrow = lax.shift_right_logical(fb, 1)
    colsel = fsel + 8 * lax.bitwise_and(fb, 1)

    def issue(g, b):
        e0 = ebase + g * CE
        pltpu.async_copy(dst_hbm.at[pl.ds(e0, CE)], dstbs[b], sems[b])

        def fill(j, c2):
            gidx[b, pl.ds(j * L, L)] = (e0 + j * L + lanes) * 8 + fbrow
            return c2

        lax.fori_loop(0, CE // L, fill, 0)
        for k in range(CE // 128):
            pltpu.async_copy(mv_hbm.at[gidx.at[b, pl.ds(k * 128, 128)]],
                             mbufs[b].at[pl.ds(k * 128, 128)], sems[b])

    def drain(b):
        pltpu.make_async_copy(dst_hbm.at[pl.ds(0, CE)], dstbs[b],
                              sems[b]).wait()
        for k in range(CE // 128):
            pltpu.make_async_copy(mv_hbm.at[gidx.at[b, pl.ds(k * 128, 128)]],
                                  mbufs[b].at[pl.ds(k * 128, 128)],
                                  sems[b]).wait()

    def process(g, b):
        dstb = dstbs[b]
        mbufg = mbufs[b]

        def group16(j, c2):
            dvec = dstb[pl.ds(j * L, L)]
            for i in range(L // 2):
                dA = dvec[2 * i]
                dB = dvec[2 * i + 1]
                dsel = dA + lanehigh * (dB - dA)
                mrow = (j * L + 2 * i) + lanehigh
                mvec = plsc.load_gather(mbufg, (mrow, colsel))
                pen = jnp.where(dA == dB, jnp.float32(0.0),
                                jnp.float32(NEG_INF))
                mswap = lax.gather(
                    mvec, swapidx[:, None],
                    dimension_numbers=lax.GatherDimensionNumbers(
                        offset_dims=(), collapsed_slice_dims=(0,),
                        start_index_map=(0,)),
                    slice_sizes=(1,),
                    mode=lax.GatherScatterMode.PROMISE_IN_BOUNDS)
                m2 = jnp.maximum(mvec, mswap + pen)
                old = plsc.load_gather(acc, (dsel, fsel))
                plsc.store_scatter(acc, (dsel, fsel), jnp.maximum(old, m2))
            return c2

        lax.fori_loop(0, CE // L, group16, 0)

    issue(0, 0)

    def chunk2(h, carry):
        for b in (0, 1):
            g = 2 * h + b

            @pl.when(g + 1 < NCHUNK)
            def _():
                issue(g + 1, 1 - b)

            drain(b)
            process(g, b)
        return carry

    lax.fori_loop(0, NCHUNK // 2, chunk2, 0)

    obase = s * (NPAD * L) + fb

    def wb(g, carry):
        def ofill(j, c2):
            oidx[g, pl.ds(j * L, L)] = obase + (g * 128 + j * L + lanes) * L
            return c2

        lax.fori_loop(0, 128 // L, ofill, 0)
        pltpu.sync_copy(acc.at[pl.ds(g * 128, 128)], part_hbm.at[oidx.at[g]])
        return carry

    lax.fori_loop(0, NPAD // 128, wb, 0)


def _sc_scatter(dst, mv):
    kern = pl.kernel(
        _scatter_body,
        out_type=jax.ShapeDtypeStruct((2 * NPAD * L, 8), jnp.float32),
        mesh=_mesh,
        scratch_types=[
            pltpu.VMEM((NPAD, 8), jnp.float32),
            pltpu.VMEM((CE,), jnp.int32),
            pltpu.VMEM((CE,), jnp.int32),
            pltpu.VMEM((CE, L), jnp.float32),
            pltpu.VMEM((CE, L), jnp.float32),
            pltpu.VMEM((2, CE), jnp.int32),
            pltpu.VMEM((NPAD // 128, 128), jnp.int32),
            pltpu.SemaphoreType.DMA,
            pltpu.SemaphoreType.DMA,
        ],
        compiler_params=pltpu.CompilerParams(use_tc_tiling_on_sc=False,
                                             needs_layout_passes=False),
    )
    return kern(dst, mv)


def _segment_max_parts(dst, m):
    mv = m.reshape(E * 8, L)
    partv = _sc_scatter(dst, mv)
    part = partv.reshape(2, NPAD, F)
    return part[0, :N], part[1, :N]


def kernel(x, edge_index, t, emb_pcs, W_enc, b_enc, W_f, Wt, bt,
           m1W1, m1b1, m1W2, m1b2, m2W1, m2b1, m2W2, m2b2,
           m3W1, m3b1, m3W2, m3b2):
    src = edge_index[0].astype(jnp.int32)
    dst = edge_index[1].astype(jnp.int32)

    def split(w1):
        w1a, w1b = w1[:3 * F], w1[3 * F:]
        c = w1a - w1b
        return ((c[:F], c[F:2 * F], c[2 * F:]),
                (w1b[:F], w1b[F:2 * F], w1b[2 * F:]))

    temb, h0 = _tc_pre(x, t, W_f.reshape(1, F // 2), Wt, bt.reshape(1, F),
                       W_enc, b_enc.reshape(1, F))

    ga, gb = h0, h0
    prep = False
    for (w1, b1, w2, b2) in ((m1W1, m1b1, m1W2, m1b2),
                             (m2W1, m2b1, m2W2, m2b2),
                             (m3W1, m3b1, m3W2, m3b2)):
        (cg, ct, ce), (bg, bt_, be) = split(w1)
        p, q = _tc_node(prep, ga, gb, temb, emb_pcs,
                        cg, ct, ce, bg, bt_, be, b1.reshape(1, F))
        pre = _sc_gather(p, q, dst, src)
        m = _tc_msg(pre, w2, b2.reshape(1, F))
        ga, gb = _segment_max_parts(dst, m)
        prep = True

    return _tc_final(ga, gb, t)


# speedup vs baseline: 2.7062x; 1.2756x over previous
import functools
import math

import jax
import jax.numpy as jnp
from jax import lax
from jax.experimental import pallas as pl
from jax.experimental.pallas import tpu as pltpu
from jax.experimental.pallas import tpu_sc as plsc

N = 10000
E = 320000
F = 128
SIGMA = 25.0

NC = 2
NS = 16
L = 16

HN = 5000
HNP = 5120
NPAD = 2 * HNP
EH = E // 2
CE = 640
NCHUNK = EH // CE
EPW = E // (NC * NS)
CG = 80
NGCHUNK = EPW // CG

_mesh = plsc.VectorSubcoreMesh(
    core_axis_name="c", subcore_axis_name="s", num_cores=NC, num_subcores=NS)

NEG_INF = float("-inf")


def _pre_body(x_ref, t_ref, wf_ref, wt_ref, bt_ref, wenc_ref, benc_ref,
              temb_ref, h0_ref):
    t = t_ref[...]
    wf = wf_ref[...]
    proj = t * wf * jnp.float32(2.0 * math.pi)
    temb0 = jnp.concatenate([jnp.sin(proj), jnp.cos(proj)], axis=-1)
    z = jnp.dot(temb0, wt_ref[...], preferred_element_type=jnp.float32)
    z = z + bt_ref[...]
    temb_ref[...] = z * jax.nn.sigmoid(z)
    h0 = jnp.dot(x_ref[...], wenc_ref[...], preferred_element_type=jnp.float32)
    h0_ref[...] = h0 + benc_ref[...]


def _node_body(prep, ga_ref, gb_ref, temb_ref, emb_ref,
               cg_ref, ct_ref, ce_ref, bg_ref, bt_ref, be_ref, b1_ref,
               p_ref, q_ref):
    if prep:
        m = jnp.maximum(ga_ref[...], gb_ref[...])
        m = jnp.where(m > NEG_INF, m, 0.0)
        g = jnp.maximum(m, 0.0)
    else:
        g = ga_ref[...]
    temb = temb_ref[...]
    emb = emb_ref[...]
    dot = lambda a, w: jnp.dot(a, w[...], preferred_element_type=jnp.float32)
    p_ref[...] = (dot(g, cg_ref) + dot(temb, ct_ref) + dot(emb, ce_ref)
                  + b1_ref[...])
    q_ref[...] = dot(g, bg_ref) + dot(temb, bt_ref) + dot(emb, be_ref)


def _msg_body(pre_ref, w2_ref, b2_ref, m_ref):
    a = jnp.maximum(pre_ref[...], 0.0)
    m_ref[...] = (jnp.dot(a, w2_ref[...], preferred_element_type=jnp.float32)
                  + b2_ref[...])


def _final_body(pa_ref, pb_ref, t_ref, out_ref):
    m = jnp.maximum(pa_ref[...], pb_ref[...])
    m = jnp.where(m > NEG_INF, m, 0.0)
    log_sigma = jnp.float32(math.log(SIGMA))
    t = t_ref[...]
    var = (jnp.exp(2.0 * t * log_sigma) - 1.0) / (2.0 * log_sigma)
    std = jnp.sqrt(var)
    out_ref[...] = m / (std + 1e-7)


_NB = 1000
_EB = 2000


def _full(shape):
    nd = len(shape)
    return pl.BlockSpec(shape, lambda i: (0,) * nd)


def _rows(block_shape):
    return pl.BlockSpec(block_shape, lambda i: (i,) + (0,) * (len(block_shape) - 1))


def _tc_pre(x, t, wf, wt, bt, wenc, benc):
    return pl.pallas_call(
        _pre_body,
        grid=(N // _NB,),
        in_specs=[_rows((_NB, F)), _rows((_NB, 1)), _full((1, F // 2)),
                  _full((F, F)), _full((1, F)), _full((F, F)), _full((1, F))],
        out_specs=[_rows((_NB, F)), _rows((_NB, F))],
        out_shape=[jax.ShapeDtypeStruct((N, F), jnp.float32),
                   jax.ShapeDtypeStruct((N, F), jnp.float32)],
    )(x, t, wf, wt, bt, wenc, benc)


def _tc_node(prep, ga, gb, temb, emb, cg, ct, ce, bg, bt, be, b1):
    return pl.pallas_call(
        functools.partial(_node_body, prep),
        grid=(N // _NB,),
        in_specs=[_rows((_NB, F))] * 4 + [_full((F, F))] * 6 + [_full((1, F))],
        out_specs=[_rows((_NB, F)), _rows((_NB, F))],
        out_shape=[jax.ShapeDtypeStruct((N, F), jnp.float32),
                   jax.ShapeDtypeStruct((N, F), jnp.float32)],
    )(ga, gb, temb, emb, cg, ct, ce, bg, bt, be, b1)


def _tc_msg(pre, w2, b2):
    return pl.pallas_call(
        _msg_body,
        grid=(E // _EB,),
        in_specs=[_rows((_EB, F)), _full((F, F)), _full((1, F))],
        out_specs=_rows((_EB, F)),
        out_shape=jax.ShapeDtypeStruct((E, F), jnp.float32),
    )(pre, w2, b2)


def _tc_final(pa, pb, t):
    return pl.pallas_call(
        _final_body,
        grid=(N // _NB,),
        in_specs=[_rows((_NB, F)), _rows((_NB, F)), _rows((_NB, 1))],
        out_specs=_rows((_NB, F)),
        out_shape=jax.ShapeDtypeStruct((N, F), jnp.float32),
    )(pa, pb, t)


def _gather_body(p_hbm, q_hbm, dst_hbm, src_hbm, pre_hbm,
                 dstb, srcb, dbuf0, dbuf1, sbuf0, sbuf1, obuf0, obuf1,
                 gsem0, gsem1, wsem0, wsem1):
    wid = lax.axis_index("s") * NC + lax.axis_index("c")
    base = wid * EPW
    pltpu.sync_copy(dst_hbm.at[pl.ds(base, EPW)], dstb)
    pltpu.sync_copy(src_hbm.at[pl.ds(base, EPW)], srcb)

    dbufs = (dbuf0, dbuf1)
    sbufs = (sbuf0, sbuf1)
    obufs = (obuf0, obuf1)
    gsems = (gsem0, gsem1)
    wsems = (wsem0, wsem1)

    def issue(g, b):
        off = g * CG
        pltpu.async_copy(p_hbm.at[dstb.at[pl.ds(off, CG)]], dbufs[b],
                         gsems[b])
        pltpu.async_copy(q_hbm.at[srcb.at[pl.ds(off, CG)]], sbufs[b],
                         gsems[b])

    def drain_gather(b):
        pltpu.make_async_copy(p_hbm.at[dstb.at[pl.ds(0, CG)]], dbufs[b],
                              gsems[b]).wait()
        pltpu.make_async_copy(q_hbm.at[srcb.at[pl.ds(0, CG)]], sbufs[b],
                              gsems[b]).wait()

    def drain_write(b):
        pltpu.make_async_copy(obufs[b], pre_hbm.at[pl.ds(base, CG), :],
                              wsems[b]).wait()

    issue(0, 0)
    issue(1, 1)

    def chunk2(h, carry):
        for b in (0, 1):
            g = 2 * h + b
            drain_gather(b)

            @pl.when(g >= 2)
            def _():
                drain_write(b)

            dbuf, sbuf, obuf = dbufs[b], sbufs[b], obufs[b]

            def row(r, c2):
                for j in range(F // L):
                    sl = pl.ds(j * L, L)
                    obuf[r, sl] = dbuf[r, sl] + sbuf[r, sl]
                return c2

            lax.fori_loop(0, CG, row, 0)
            pltpu.async_copy(obuf, pre_hbm.at[pl.ds(base + g * CG, CG), :],
                             wsems[b])

            @pl.when(g + 2 < NGCHUNK)
            def _():
                issue(g + 2, b)
        return carry

    lax.fori_loop(0, NGCHUNK // 2, chunk2, 0)

    if NGCHUNK % 2:
        gl = NGCHUNK - 1
        drain_gather(0)
        drain_write(0)

        def rowl(r, c2):
            for j in range(F // L):
                sl = pl.ds(j * L, L)
                obuf0[r, sl] = dbuf0[r, sl] + sbuf0[r, sl]
            return c2

        lax.fori_loop(0, CG, rowl, 0)
        pltpu.sync_copy(obuf0, pre_hbm.at[pl.ds(base + gl * CG, CG), :])
        drain_write(1)
    else:
        drain_write(0)
        drain_write(1)


def _sc_gather(p, q, dst, src):
    kern = pl.kernel(
        _gather_body,
        out_type=jax.ShapeDtypeStruct((E, F), jnp.float32),
        mesh=_mesh,
        scratch_types=[
            pltpu.VMEM((EPW,), jnp.int32),
            pltpu.VMEM((EPW,), jnp.int32),
            pltpu.VMEM((CG, F), jnp.float32),
            pltpu.VMEM((CG, F), jnp.float32),
            pltpu.VMEM((CG, F), jnp.float32),
            pltpu.VMEM((CG, F), jnp.float32),
            pltpu.VMEM((CG, F), jnp.float32),
            pltpu.VMEM((CG, F), jnp.float32),
            pltpu.SemaphoreType.DMA,
            pltpu.SemaphoreType.DMA,
            pltpu.SemaphoreType.DMA,
            pltpu.SemaphoreType.DMA,
        ],
    )
    return kern(p, q, dst, src)


def _scatter_body(dst_hbm, mv_hbm, part_hbm, acc,
                  dstb0, dstb1, mbuf0, mbuf1, gidx, oidx, sem0, sem1):
    s = lax.axis_index("c")
    fb = lax.axis_index("s")
    lanes = lax.iota(jnp.int32, L)
    fsel = jnp.bitwise_and(lanes, 7)
    lanehigh = jnp.right_shift(lanes, 3)
    swapidx = jnp.bitwise_xor(lanes, 8)

    neg = jnp.full((L,), NEG_INF, jnp.float32)

    def initrow(r, c):
        plsc.store_scatter(acc, (2 * r + lanehigh, fsel), neg)
        return c

    lax.fori_loop(0, NPAD // 2, initrow, 0)

    ebase = s * EH
    dstbs = (dstb0, dstb1)
    mbufs = (mbuf0, mbuf1)
    sems = (sem0, sem1)

    fbrow = lax.shift_right_logical(fb, 1)
    colsel = fsel + 8 * lax.bitwise_and(fb, 1)

    def issue(g, b):
        e0 = ebase + g * CE
        pltpu.async_copy(dst_hbm.at[pl.ds(e0, CE)], dstbs[b], sems[b])

        def fill(j, c2):
            gidx[b, pl.ds(j * L, L)] = (e0 + j * L + lanes) * 8 + fbrow
            return c2

        lax.fori_loop(0, CE // L, fill, 0)
        for k in range(CE // 128):
            pltpu.async_copy(mv_hbm.at[gidx.at[b, pl.ds(k * 128, 128)]],
                             mbufs[b].at[pl.ds(k * 128, 128)], sems[b])

    def drain(b):
        pltpu.make_async_copy(dst_hbm.at[pl.ds(0, CE)], dstbs[b],
                              sems[b]).wait()
        for k in range(CE // 128):
            pltpu.make_async_copy(mv_hbm.at[gidx.at[b, pl.ds(k * 128, 128)]],
                                  mbufs[b].at[pl.ds(k * 128, 128)],
                                  sems[b]).wait()

    def process(g, b):
        dstb = dstbs[b]
        mbufg = mbufs[b]

        def group16(j, c2):
            dvec = dstb[pl.ds(j * L, L)]
            for i in range(L // 2):
                dA = dvec[2 * i]
                dB = dvec[2 * i + 1]
                dsel = dA + lanehigh * (dB - dA)
                mrow = (j * L + 2 * i) + lanehigh
                mvec = plsc.load_gather(mbufg, (mrow, colsel))
                pen = jnp.where(dA == dB, jnp.float32(0.0),
                                jnp.float32(NEG_INF))
                mswap = lax.gather(
                    mvec, swapidx[:, None],
                    dimension_numbers=lax.GatherDimensionNumbers(
                        offset_dims=(), collapsed_slice_dims=(0,),
                        start_index_map=(0,)),
                    slice_sizes=(1,),
                    mode=lax.GatherScatterMode.PROMISE_IN_BOUNDS)
                m2 = jnp.maximum(mvec, mswap + pen)
                old = plsc.load_gather(acc, (dsel, fsel))
                plsc.store_scatter(acc, (dsel, fsel), jnp.maximum(old, m2))
            return c2

        lax.fori_loop(0, CE // L, group16, 0)

    issue(0, 0)

    def chunk2(h, carry):
        for b in (0, 1):
            g = 2 * h + b

            @pl.when(g + 1 < NCHUNK)
            def _():
                issue(g + 1, 1 - b)

            drain(b)
            process(g, b)
        return carry

    lax.fori_loop(0, NCHUNK // 2, chunk2, 0)

    obase = s * (NPAD * L) + fb

    def wb(g, carry):
        def ofill(j, c2):
            oidx[g, pl.ds(j * L, L)] = obase + (g * 128 + j * L + lanes) * L
            return c2

        lax.fori_loop(0, 128 // L, ofill, 0)
        pltpu.sync_copy(acc.at[pl.ds(g * 128, 128)], part_hbm.at[oidx.at[g]])
        return carry

    lax.fori_loop(0, NPAD // 128, wb, 0)


def _sc_scatter(dst, mv):
    kern = pl.kernel(
        _scatter_body,
        out_type=jax.ShapeDtypeStruct((2 * NPAD * L, 8), jnp.float32),
        mesh=_mesh,
        scratch_types=[
            pltpu.VMEM((NPAD, 8), jnp.float32),
            pltpu.VMEM((CE,), jnp.int32),
            pltpu.VMEM((CE,), jnp.int32),
            pltpu.VMEM((CE, L), jnp.float32),
            pltpu.VMEM((CE, L), jnp.float32),
            pltpu.VMEM((2, CE), jnp.int32),
            pltpu.VMEM((NPAD // 128, 128), jnp.int32),
            pltpu.SemaphoreType.DMA,
            pltpu.SemaphoreType.DMA,
        ],
        compiler_params=pltpu.CompilerParams(use_tc_tiling_on_sc=False,
                                             needs_layout_passes=False),
    )
    return kern(dst, mv)


def _segment_max_parts(dst, m):
    mv = m.reshape(E * 8, L)
    partv = _sc_scatter(dst, mv)
    part = partv.reshape(2, NPAD, F)
    return part[0, :N], part[1, :N]


def kernel(x, edge_index, t, emb_pcs, W_enc, b_enc, W_f, Wt, bt,
           m1W1, m1b1, m1W2, m1b2, m2W1, m2b1, m2W2, m2b2,
           m3W1, m3b1, m3W2, m3b2):
    src = edge_index[0].astype(jnp.int32)
    dst = edge_index[1].astype(jnp.int32)

    def split(w1):
        w1a, w1b = w1[:3 * F], w1[3 * F:]
        c = w1a - w1b
        return ((c[:F], c[F:2 * F], c[2 * F:]),
                (w1b[:F], w1b[F:2 * F], w1b[2 * F:]))

    temb, h0 = _tc_pre(x, t, W_f.reshape(1, F // 2), Wt, bt.reshape(1, F),
                       W_enc, b_enc.reshape(1, F))

    ga, gb = h0, h0
    prep = False
    for (w1, b1, w2, b2) in ((m1W1, m1b1, m1W2, m1b2),
                             (m2W1, m2b1, m2W2, m2b2),
                             (m3W1, m3b1, m3W2, m3b2)):
        (cg, ct, ce), (bg, bt_, be) = split(w1)
        p, q = _tc_node(prep, ga, gb, temb, emb_pcs,
                        cg, ct, ce, bg, bt_, be, b1.reshape(1, F))
        pre = _sc_gather(p, q, dst, src)
        m = _tc_msg(pre, w2, b2.reshape(1, F))
        ga, gb = _segment_max_parts(dst, m)
        prep = True

    return _tc_final(ga, gb, t)
